# baseline (speedup 1.0000x reference)
import functools

import jax
import jax.numpy as jnp
from jax import lax
from jax.experimental import pallas as pl
from jax.experimental.pallas import tpu as pltpu
from jax.experimental.pallas import tpu_sc as plsc

_N, _D, _K = 4096, 128, 3
_B = 256
_KP = 8
_NC, _NS = 2, 16
_NW = _NC * _NS
_NH = 2
_R = _N // _NH // _NW
_H = _D // 16
def _topk_body(x_ref, yt_ref, idx_ref):
    x = x_ref[...]
    yt = yt_ref[...]
    sq_x = jnp.sum(x * x, axis=1, keepdims=True)
    sq_y = jnp.sum(yt * yt, axis=0, keepdims=True)
    d2 = sq_x + sq_y - 2.0 * jnp.dot(x, yt, preferred_element_type=jnp.float32)

    rows = lax.broadcasted_iota(jnp.int32, (_N, _B), 0)
    key = (lax.bitcast_convert_type(d2, jnp.int32) & ~jnp.int32(0xFFF)) | rows

    idxs = []
    for _ in range(_K):
        m = jnp.min(key, axis=0, keepdims=True)
        idxs.append(m & jnp.int32(0xFFF))
        key = jnp.where(key == m, jnp.int32(0x7FFFFFFF), key)
    pad = jnp.zeros((_KP - _K, _B), jnp.int32)
    idx_ref[...] = jnp.concatenate(idxs + [pad], axis=0)


def _topk(x, xt, half):
    nh = _N // _NH
    off = half * (nh // _B)
    return pl.pallas_call(
        _topk_body,
        grid=(nh // _B,),
        in_specs=[
            pl.BlockSpec((_N, _D), lambda i: (0, 0)),
            pl.BlockSpec((_D, _B), lambda i: (0, i + off)),
        ],
        out_specs=pl.BlockSpec((_KP, _B), lambda i: (0, i)),
        out_shape=jax.ShapeDtypeStruct((_KP, nh), jnp.int32),
    )(x, xt)


def _sc_body(half, x_hbm, idxt_hbm, out_hbm,
             xq, g0, g1, g2, i0, i1, i2, out_v, sem):
    wid = lax.axis_index("s") * _NC + lax.axis_index("c")
    base = wid * _R
    pltpu.sync_copy(x_hbm.at[pl.ds(half * (_N // _NH) + base, _R)], xq)
    pltpu.sync_copy(idxt_hbm.at[0, wid], i0)
    pltpu.sync_copy(idxt_hbm.at[1, wid], i1)
    pltpu.sync_copy(idxt_hbm.at[2, wid], i2)
    c0 = pltpu.async_copy(x_hbm.at[i0], g0, sem)
    c1 = pltpu.async_copy(x_hbm.at[i1], g1, sem)
    c2 = pltpu.async_copy(x_hbm.at[i2], g2, sem)
    c0.wait()
    c1.wait()
    c2.wait()

    def row(r, carry):
        ws = []
        for g in (g0, g1, g2):
            acc = jnp.zeros((16,), jnp.float32)
            for h in range(_H):
                v = g[r, pl.ds(h * 16, 16)] - xq[r, pl.ds(h * 16, 16)]
                acc = acc + v * v
            d2 = acc[0]
            for t in range(1, 16):
                d2 = d2 + acc[t]
            d2v = jnp.broadcast_to(d2, (16,))
            ws.append(1.0 / jnp.maximum(d2v, 1e-16))
        w0, w1, w2 = ws
        inv = 1.0 / (w0 + w1 + w2)
        for h in range(_H):
            s = pl.ds(h * 16, 16)
            out_v[r, s] = (w0 * g0[r, s] + w1 * g1[r, s] + w2 * g2[r, s]) * inv
        return carry

    lax.fori_loop(0, _R, row, 0)
    pltpu.sync_copy(out_v, out_hbm.at[pl.ds(base, _R)])


def _sc_interpolate(x, idx_t, half):
    mesh = plsc.VectorSubcoreMesh(core_axis_name="c", subcore_axis_name="s")
    run = functools.partial(
        pl.kernel,
        out_type=jax.ShapeDtypeStruct((_N // _NH, _D), jnp.float32),
        mesh=mesh,
        scratch_types=[
            pltpu.VMEM((_R, _D), jnp.float32),
            pltpu.VMEM((_R, _D), jnp.float32),
            pltpu.VMEM((_R, _D), jnp.float32),
            pltpu.VMEM((_R, _D), jnp.float32),
            pltpu.VMEM((_R,), jnp.int32),
            pltpu.VMEM((_R,), jnp.int32),
            pltpu.VMEM((_R,), jnp.int32),
            pltpu.VMEM((_R, _D), jnp.float32),
            pltpu.SemaphoreType.DMA,
        ],
    )(functools.partial(_sc_body, half))
    return run(x, idx_t)


def kernel(x):
    xt = x.T
    outs = []
    for h in range(_NH):
        idx = _topk(x, xt, h)
        idx_t = idx.reshape(_KP, _NW, _R)
        outs.append(_sc_interpolate(x, idx_t, h))
    return jnp.concatenate(outs, axis=0)


# speedup vs baseline: 1.0097x; 1.0097x over previous
import functools

import jax
import jax.numpy as jnp
from jax import lax
from jax.experimental import pallas as pl
from jax.experimental.pallas import tpu as pltpu
from jax.experimental.pallas import tpu_sc as plsc

_N, _D, _K = 4096, 128, 3
_B = 256
_KP = 8
_NC, _NS = 2, 16
_NW = _NC * _NS
_NH = 1
_R = _N // _NH // _NW
_H = _D // 16
def _topk_body(x_ref, y_ref, idx_ref):
    x = x_ref[...]
    y = y_ref[...]
    hsq_x = 0.5 * jnp.sum(x * x, axis=1, keepdims=True)
    e = hsq_x - lax.dot_general(x, y, (((1,), (1,)), ((), ())),
                                preferred_element_type=jnp.float32)
    b = lax.bitcast_convert_type(e, jnp.int32)
    flip = lax.shift_right_logical(lax.shift_right_arithmetic(b, 31), 1)
    rows = lax.broadcasted_iota(jnp.int32, (_N, _B), 0)
    key = ((b ^ flip) & ~jnp.int32(0xFFF)) | rows

    idxs = []
    for _ in range(_K):
        m = jnp.min(key, axis=0, keepdims=True)
        idxs.append(m & jnp.int32(0xFFF))
        key = jnp.where(key == m, jnp.int32(0x7FFFFFFF), key)
    pad = jnp.zeros((_KP - _K, _B), jnp.int32)
    idx_ref[...] = jnp.concatenate(idxs + [pad], axis=0)


def _topk(x):
    return pl.pallas_call(
        _topk_body,
        grid=(_N // _B,),
        in_specs=[
            pl.BlockSpec((_N, _D), lambda i: (0, 0)),
            pl.BlockSpec((_B, _D), lambda i: (i, 0)),
        ],
        out_specs=pl.BlockSpec((_KP, _B), lambda i: (0, i)),
        out_shape=jax.ShapeDtypeStruct((_KP, _N), jnp.int32),
    )(x, x)


def _sc_body(half, x_hbm, idxt_hbm, out_hbm,
             xq, g0, g1, g2, i0, i1, i2, out_v, sem):
    wid = lax.axis_index("s") * _NC + lax.axis_index("c")
    base = wid * _R
    pltpu.sync_copy(x_hbm.at[pl.ds(half * (_N // _NH) + base, _R)], xq)
    pltpu.sync_copy(idxt_hbm.at[0, wid], i0)
    pltpu.sync_copy(idxt_hbm.at[1, wid], i1)
    pltpu.sync_copy(idxt_hbm.at[2, wid], i2)
    c0 = pltpu.async_copy(x_hbm.at[i0], g0, sem)
    c1 = pltpu.async_copy(x_hbm.at[i1], g1, sem)
    c2 = pltpu.async_copy(x_hbm.at[i2], g2, sem)
    c0.wait()
    c1.wait()
    c2.wait()

    def row(r, carry):
        ws = []
        for g in (g0, g1, g2):
            acc = jnp.zeros((16,), jnp.float32)
            for h in range(_H):
                v = g[r, pl.ds(h * 16, 16)] - xq[r, pl.ds(h * 16, 16)]
                acc = acc + v * v
            d2 = acc[0]
            for t in range(1, 16):
                d2 = d2 + acc[t]
            d2v = jnp.broadcast_to(d2, (16,))
            ws.append(1.0 / jnp.maximum(d2v, 1e-16))
        w0, w1, w2 = ws
        inv = 1.0 / (w0 + w1 + w2)
        for h in range(_H):
            s = pl.ds(h * 16, 16)
            out_v[r, s] = (w0 * g0[r, s] + w1 * g1[r, s] + w2 * g2[r, s]) * inv
        return carry

    lax.fori_loop(0, _R, row, 0)
    pltpu.sync_copy(out_v, out_hbm.at[pl.ds(base, _R)])


def _sc_interpolate(x, idx_t, half):
    mesh = plsc.VectorSubcoreMesh(core_axis_name="c", subcore_axis_name="s")
    run = functools.partial(
        pl.kernel,
        out_type=jax.ShapeDtypeStruct((_N // _NH, _D), jnp.float32),
        mesh=mesh,
        scratch_types=[
            pltpu.VMEM((_R, _D), jnp.float32),
            pltpu.VMEM((_R, _D), jnp.float32),
            pltpu.VMEM((_R, _D), jnp.float32),
            pltpu.VMEM((_R, _D), jnp.float32),
            pltpu.VMEM((_R,), jnp.int32),
            pltpu.VMEM((_R,), jnp.int32),
            pltpu.VMEM((_R,), jnp.int32),
            pltpu.VMEM((_R, _D), jnp.float32),
            pltpu.SemaphoreType.DMA,
        ],
    )(functools.partial(_sc_body, half))
    return run(x, idx_t)


def kernel(x):
    idx = _topk(x)
    idx_t = idx.reshape(_KP, _NW, _R)
    return _sc_interpolate(x, idx_t, 0)
